# baseline (speedup 1.0000x reference)
import functools

import jax
import jax.numpy as jnp
from jax import lax
from jax.experimental import pallas as pl
from jax.experimental.pallas import tpu as pltpu
from jax.experimental.pallas import tpu_sc as plsc

VOCAB = 1000000
D = 64
SEQ = 200
BATCH = 4096
R = BATCH * SEQ

NC, NS, L = 2, 16, 16
NW = NC * NS

K_SEQ = 4
CHUNK_ROWS = K_SEQ * SEQ
G_ROWS = 100
N_GATHER = CHUNK_ROWS // G_ROWS
N_CHUNKS = R // (NW * CHUNK_ROWS)
D_CH = D // L


def _positional_encoding():
    i = jnp.arange(0, D, 2) / D
    pos = jnp.arange(0, SEQ)[:, None].astype(jnp.float32)
    angle_freq = jnp.exp(i * -jnp.log(jnp.array(10000.0)))
    out = jnp.zeros((SEQ, D), dtype=jnp.float32)
    out = out.at[:, 0::2].set(jnp.sin(pos * angle_freq))
    out = out.at[:, 1::2].set(jnp.cos(pos * angle_freq))
    return out


def _sc_embed(idx2, table, pos):
    mesh = plsc.VectorSubcoreMesh(core_axis_name="c", subcore_axis_name="s")

    @functools.partial(
        pl.kernel,
        out_type=jax.ShapeDtypeStruct((R, D), jnp.float32),
        mesh=mesh,
        compiler_params=pltpu.CompilerParams(use_tc_tiling_on_sc=False),
        scratch_types=[
            pltpu.VMEM((N_GATHER, G_ROWS), jnp.int32),
            pltpu.VMEM((CHUNK_ROWS, D), jnp.float32),
            pltpu.VMEM((SEQ, D), jnp.float32),
            pltpu.SemaphoreType.DMA,
        ],
    )
    def body(idx_hbm, table_hbm, pos_hbm, out_hbm, idx_v, rows_v, pos_v, gsem):
        wid = lax.axis_index("s") * NC + lax.axis_index("c")
        pltpu.sync_copy(pos_hbm, pos_v)

        def chunk_body(c, carry):
            row_base = (wid * N_CHUNKS + c) * CHUNK_ROWS
            pltpu.sync_copy(
                idx_hbm.at[pl.ds(pl.multiple_of(row_base // G_ROWS, 8),
                                 N_GATHER)], idx_v)
            cps = [
                pltpu.async_copy(
                    table_hbm.at[idx_v.at[i]],
                    rows_v.at[pl.ds(i * G_ROWS, G_ROWS)],
                    gsem,
                )
                for i in range(N_GATHER)
            ]
            for cp in cps:
                cp.wait()

            def add_body(t, carry2):
                p = t // D_CH
                j = (t % D_CH) * L
                pv = pos_v[p, pl.ds(j, L)]
                for s in range(K_SEQ):
                    r = s * SEQ + p
                    rows_v[r, pl.ds(j, L)] = rows_v[r, pl.ds(j, L)] + pv
                return carry2

            lax.fori_loop(0, SEQ * D_CH, add_body, 0, unroll=2)
            pltpu.sync_copy(rows_v, out_hbm.at[pl.ds(row_base, CHUNK_ROWS)])
            return carry

        lax.fori_loop(0, N_CHUNKS, chunk_body, 0)

    return body(idx2, table, pos)


def kernel(inputs, table):
    idx2 = inputs.astype(jnp.int32).reshape(R // G_ROWS, G_ROWS)
    pos = _positional_encoding()
    out = _sc_embed(idx2, table, pos)
    return out.reshape(BATCH, SEQ, D)


# speedup vs baseline: 1.3507x; 1.3507x over previous
import functools

import jax
import jax.numpy as jnp
from jax import lax
from jax.experimental import pallas as pl
from jax.experimental.pallas import tpu as pltpu
from jax.experimental.pallas import tpu_sc as plsc

VOCAB = 1000000
D = 64
SEQ = 200
BATCH = 4096
L = 16
NC, NS = 2, 16
NW = NC * NS

K_SEQ = 4
G_SPLIT = ((0, 104), (104, 96))
SEQ_PER_W = BATCH // NW
N_CHUNKS = SEQ_PER_W // K_SEQ
N_GROUPS = N_CHUNKS // 2
D_CH = D // L


def _positional_encoding():
    i = jnp.arange(0, D, 2) / D
    pos = jnp.arange(0, SEQ)[:, None].astype(jnp.float32)
    angle_freq = jnp.exp(i * -jnp.log(jnp.array(10000.0)))
    out = jnp.zeros((SEQ, D), dtype=jnp.float32)
    out = out.at[:, 0::2].set(jnp.sin(pos * angle_freq))
    out = out.at[:, 1::2].set(jnp.cos(pos * angle_freq))
    return out


def _sc_embed(idx, table, pos):
    mesh = plsc.VectorSubcoreMesh(core_axis_name="c", subcore_axis_name="s")

    @functools.partial(
        pl.kernel,
        out_type=jax.ShapeDtypeStruct((BATCH, SEQ, D), jnp.float32),
        mesh=mesh,
        compiler_params=pltpu.CompilerParams(use_tc_tiling_on_sc=False),
        scratch_types=[
            pltpu.VMEM((2, 8, SEQ), jnp.int32),
            pltpu.VMEM((2, K_SEQ, SEQ, D), jnp.float32),
            pltpu.VMEM((SEQ, D), jnp.float32),
            pltpu.SemaphoreType.DMA,
            pltpu.SemaphoreType.DMA,
            pltpu.SemaphoreType.DMA,
            pltpu.SemaphoreType.DMA,
        ],
    )
    def body(idx_hbm, table_hbm, pos_hbm, out_hbm,
             idx_v, rows_v, pos_v, gsem0, gsem1, osem0, osem1):
        wid = lax.axis_index("s") * NC + lax.axis_index("c")
        seq0 = wid * SEQ_PER_W
        gsems = (gsem0, gsem1)
        osems = (osem0, osem1)
        pltpu.sync_copy(pos_hbm, pos_v)

        def stage_idx(g):
            base = pl.multiple_of(seq0 + g * 8, 8)
            pltpu.sync_copy(idx_hbm.at[pl.ds(base, 8)], idx_v.at[lax.rem(g, 2)])

        def fire_gathers(c, nb):
            pg = lax.rem(c // 2, 2)
            ls = lax.rem(c, 2) * K_SEQ
            for s in range(K_SEQ):
                for off, sz in G_SPLIT:
                    pltpu.async_copy(
                        table_hbm.at[idx_v.at[pg, ls + s, pl.ds(off, sz)]],
                        rows_v.at[nb, s, pl.ds(off, sz)],
                        gsems[nb],
                    )

        def chunk_slice(c):
            return pl.ds(pl.multiple_of(seq0 + c * K_SEQ, K_SEQ), K_SEQ)

        def handle(c, nb):
            pltpu.make_async_copy(
                out_hbm.at[chunk_slice(c)], rows_v.at[nb], gsems[nb]).wait()

            for j in range(D_CH):
                jo = j * L

                def add_body(p, carry):
                    pv = pos_v[p, pl.ds(jo, L)]
                    for s in range(K_SEQ):
                        rows_v[nb, s, p, pl.ds(jo, L)] = (
                            rows_v[nb, s, p, pl.ds(jo, L)] + pv)
                    return carry

                lax.fori_loop(0, SEQ, add_body, 0, unroll=2)

            @pl.when(c > 0)
            def _():
                pltpu.make_async_copy(
                    rows_v.at[1 - nb], out_hbm.at[chunk_slice(c - 1)],
                    osems[1 - nb]).wait()

            pltpu.async_copy(rows_v.at[nb], out_hbm.at[chunk_slice(c)],
                             osems[nb])

            @pl.when(c + 1 < N_CHUNKS)
            def _():
                @pl.when(lax.rem(c + 1, 2) == 0)
                def _():
                    stage_idx((c + 1) // 2)
                fire_gathers(c + 1, 1 - nb)

        stage_idx(0)
        fire_gathers(0, 0)

        def pair_body(g, carry):
            handle(2 * g, 0)
            handle(2 * g + 1, 1)
            return carry

        lax.fori_loop(0, N_GROUPS, pair_body, 0)

        pltpu.make_async_copy(
            rows_v.at[1], out_hbm.at[chunk_slice(N_CHUNKS - 1)], osem1).wait()

    return body(idx, table, pos)


def kernel(inputs, table):
    idx = inputs.astype(jnp.int32)
    pos = _positional_encoding()
    return _sc_embed(idx, table, pos)
